# baseline (speedup 1.0000x reference)
import jax
import jax.numpy as jnp
from jax import lax
from jax.experimental import pallas as pl
from jax.experimental.pallas import tpu as pltpu
from jax.experimental.pallas import tpu_sc as plsc

N_D, E_D = 50000, 800000
N_R, E_R = 10000, 320000
H = 64
BR = 1024
NB_D, NB_R = 49, 10
NDP = NB_D * BR
NRP = NB_R * BR
CH = 128
NTILES = 16
DCHP = 6400
RCHP = 2560

_mesh = lambda: plsc.VectorSubcoreMesh(core_axis_name="c", subcore_axis_name="s",
                                       num_cores=2, num_subcores=16)
_SC_PARAMS = pltpu.CompilerParams(use_tc_tiling_on_sc=False)


def _zero_fill(slab_v, srows):
    z = jnp.zeros((16,), jnp.float32)
    width = slab_v.shape[1]

    def zb(i, _):
        for k in range(8):
            for c0 in range(0, width, 16):
                slab_v[i * 8 + k, c0:c0 + 16] = z
        return _

    lax.fori_loop(0, srows // 8, zb, None)


def _sc_seg(width, kb, phases, ntab, ne):
    max_pad = max(p["n_pad"] for p in phases)
    max_srows = max_pad // NTILES // 2
    out_rows = sum(2 * p["n_pad"] for p in phases)
    scratch = ([
        pltpu.VMEM((2, kb, 2, CH), jnp.int32),
        pltpu.VMEM((2, kb, CH, width), jnp.float32),
        pltpu.VMEM((max_srows, width), jnp.float32),
        pltpu.VMEM_SHARED((max_pad, width), jnp.float32),
    ] + [pltpu.SemaphoreType.DMA] * (kb + 2))

    def body(*args):
        tabs = args[:ntab]
        es = args[ntab:ntab + ne]
        out = args[ntab + ne]
        idx_v, rows_v, slab_v, acc = args[ntab + ne + 1:ntab + ne + 5]
        sems = args[ntab + ne + 5:]
        gsem, ssem = sems[:kb], sems[kb:]
        c = lax.axis_index("c")
        s = lax.axis_index("s")

        off = 0
        for ph in phases:
            n_pad, bpt = ph["n_pad"], ph["bpt"]
            gi, si = (0, 1) if ph["fwd"] else (1, 0)
            ta, tb = tabs[ph["ia"]], tabs[ph["ib"]]
            e_hbm = es[ph["ei"]]
            rows_per_tile = n_pad // NTILES
            srows = rows_per_tile // 2
            row0 = s * rows_per_tile
            _zero_fill(slab_v, srows)
            for h in range(2):
                pltpu.sync_copy(slab_v.at[pl.ds(0, srows)],
                                acc.at[pl.ds(row0 + h * srows, srows), :])
            plsc.subcore_barrier()
            if ph["seg16"]:
                base = (s * 2 + c) * bpt * kb
            else:
                base = s * bpt * kb

            def blockwork(b, table):
                gd = [pltpu.async_copy(table.at[idx_v.at[b, k, gi]],
                                       rows_v.at[b, k], gsem[k])
                      for k in range(kb)]
                for k in range(kb):
                    gd[k].wait()
                    pltpu.async_copy(rows_v.at[b, k], acc.at[idx_v.at[b, k, si]],
                                     ssem[b], add=True)

            def pair(j, carry):
                for b in range(2):
                    t = j * 2 + b

                    @pl.when(t >= 2)
                    def _drain():
                        for k in range(kb):
                            pltpu.make_async_copy(ta.at[pl.ds(0, CH)],
                                                  rows_v.at[b, k],
                                                  ssem[b]).wait()

                    pltpu.sync_copy(e_hbm.at[pl.ds(base + t * kb, kb)],
                                    idx_v.at[b])
                    if ph["seg16"]:
                        blockwork(b, ta)
                    else:
                        @pl.when(c == 0)
                        def _g0():
                            blockwork(b, ta)

                        @pl.when(c == 1)
                        def _g1():
                            blockwork(b, tb)
                return carry

            lax.fori_loop(0, bpt // 2, pair, None)
            for b in range(2):
                for k in range(kb):
                    pltpu.make_async_copy(ta.at[pl.ds(0, CH)],
                                          rows_v.at[b, k], ssem[b]).wait()
            plsc.subcore_barrier()
            obase = off + c * n_pad + row0
            for h in range(2):
                pltpu.sync_copy(acc.at[pl.ds(row0 + h * srows, srows), :],
                                slab_v.at[pl.ds(0, srows)])
                pltpu.sync_copy(slab_v.at[pl.ds(0, srows)],
                                out.at[pl.ds(obase + h * srows, srows), :])
            off += 2 * n_pad

    return pl.kernel(body,
                     out_type=jax.ShapeDtypeStruct((out_rows, width),
                                                   jnp.float32),
                     mesh=_mesh(), scratch_types=scratch,
                     compiler_params=_SC_PARAMS)


def _rowmask(i, n_nodes):
    rows = i * BR + lax.broadcasted_iota(jnp.int32, (BR, 1), 0)
    return rows < n_nodes


def _stats_accum(i, st_out, p):
    st = jnp.concatenate([jnp.sum(p, 0, keepdims=True),
                          jnp.sum(p * p, 0, keepdims=True)], 0)

    @pl.when(i == 0)
    def _():
        st_out[...] = st

    @pl.when(i > 0)
    def _():
        st_out[...] = st_out[...] + st


def _tc_pre1(n_nodes, nb, deg_col, two_dir, soff):
    def kern(*args):
        if two_dir:
            (sf0, sf1, sb0, sb1, x, wlf, wlb, wr, b, p_out, st_out) = args
        else:
            (sf0, sf1, x, wlf, wr, b, p_out, st_out) = args
        i = pl.program_id(0)
        sfb = sf0[...] + sf1[...]
        mf = sfb / jnp.maximum(sfb[:, deg_col:deg_col + 1], 1.0)
        p = jnp.dot(mf, wlf[...], preferred_element_type=jnp.float32)
        if two_dir:
            sbb = sb0[...] + sb1[...]
            mb = sbb / jnp.maximum(sbb[:, deg_col:deg_col + 1], 1.0)
            p = p + jnp.dot(mb, wlb[...], preferred_element_type=jnp.float32)
        p = p + jnp.dot(x[...], wr[...], preferred_element_type=jnp.float32)
        p = p + b[...]
        p = jnp.where(_rowmask(i, n_nodes), p, 0.0)
        p_out[...] = p
        _stats_accum(i, st_out, p)

    n_pad = nb * BR
    half = lambda j: pl.BlockSpec((BR, 16),
                                  lambda i, j=j: (i + soff + j * nb, 0))
    full16 = pl.BlockSpec((16, 64), lambda i: (0, 0))
    in_specs = [half(0), half(1)]
    if two_dir:
        in_specs += [half(2), half(3)]
    in_specs += [pl.BlockSpec((BR, 16), lambda i: (i, 0)), full16]
    if two_dir:
        in_specs += [full16]
    in_specs += [full16, pl.BlockSpec((1, 64), lambda i: (0, 0))]
    return pl.pallas_call(
        kern, grid=(nb,), in_specs=in_specs,
        out_specs=[pl.BlockSpec((BR, 64), lambda i: (i, 0)),
                   pl.BlockSpec((2, 64), lambda i: (0, 0))],
        out_shape=[jax.ShapeDtypeStruct((n_pad, 64), jnp.float32),
                   jax.ShapeDtypeStruct((2, 64), jnp.float32)])


def _tc_pre2(n_nodes, nb, deg_col, two_dir, nsplit, s2off, s1off):
    width = 64 // nsplit

    def kern(*args):
        args = list(args)
        s2f = [args.pop(0) for _ in range(nsplit)]
        s2b = [args.pop(0) for _ in range(nsplit)] if two_dir else None
        hs = [args.pop(0) for _ in range(nsplit)]
        sf0, sf1 = args.pop(0), args.pop(0)
        sb = (args.pop(0), args.pop(0)) if two_dir else None
        if two_dir:
            wlf, wlb, wr, b, p_out, st_out = args
        else:
            wlf, wr, b, p_out, st_out = args
        i = pl.program_id(0)
        degf = jnp.maximum(sf0[:, deg_col:deg_col + 1]
                           + sf1[:, deg_col:deg_col + 1], 1.0)
        m2f = jnp.concatenate([r[...] for r in s2f], axis=1) / degf
        p = jnp.dot(m2f, wlf[...], preferred_element_type=jnp.float32)
        if two_dir:
            degb = jnp.maximum(sb[0][:, deg_col:deg_col + 1]
                               + sb[1][:, deg_col:deg_col + 1], 1.0)
            m2b = jnp.concatenate([r[...] for r in s2b], axis=1) / degb
            p = p + jnp.dot(m2b, wlb[...], preferred_element_type=jnp.float32)
        hcat = jnp.concatenate([r[...] for r in hs], axis=1)
        p = p + jnp.dot(hcat, wr[...], preferred_element_type=jnp.float32)
        p = p + b[...]
        p = jnp.where(_rowmask(i, n_nodes), p, 0.0)
        p_out[...] = p
        _stats_accum(i, st_out, p)

    n_pad = nb * BR
    strip = lambda j: pl.BlockSpec((BR, width),
                                   lambda i, j=j: (i + s2off + j * nb, 0))
    s16 = lambda j: pl.BlockSpec((BR, 16),
                                 lambda i, j=j: (i + s1off + j * nb, 0))
    hstrip = pl.BlockSpec((BR, width), lambda i: (i, 0))
    full64 = pl.BlockSpec((64, 64), lambda i: (0, 0))
    in_specs = [strip(j) for j in range(nsplit)]
    if two_dir:
        in_specs += [strip(nsplit + j) for j in range(nsplit)]
    in_specs += [hstrip] * nsplit
    in_specs += [s16(0), s16(1)]
    if two_dir:
        in_specs += [s16(2), s16(3)]
    in_specs += [full64]
    if two_dir:
        in_specs += [full64]
    in_specs += [full64, pl.BlockSpec((1, 64), lambda i: (0, 0))]
    return pl.pallas_call(
        kern, grid=(nb,), in_specs=in_specs,
        out_specs=[pl.BlockSpec((BR, 64), lambda i: (i, 0)),
                   pl.BlockSpec((2, 64), lambda i: (0, 0))],
        out_shape=[jax.ShapeDtypeStruct((n_pad, 64), jnp.float32),
                   jax.ShapeDtypeStruct((2, 64), jnp.float32)])


def _tc_bnrelu(n_nodes, nb, nsplit):
    width = 64 // nsplit

    def kern(*args):
        p, st, g, b = args[:4]
        outs = args[4:]
        mu = st[0:1, :] * (1.0 / n_nodes)
        var = st[1:2, :] * (1.0 / n_nodes) - mu * mu
        scale = g[...] * lax.rsqrt(var + 1e-5)
        h = jnp.maximum((p[...] - mu) * scale + b[...], 0.0)
        for j, o in enumerate(outs):
            o[...] = h[:, j * width:(j + 1) * width]

    n_pad = nb * BR
    return pl.pallas_call(
        kern, grid=(nb,),
        in_specs=[pl.BlockSpec((BR, 64), lambda i: (i, 0)),
                  pl.BlockSpec((2, 64), lambda i: (0, 0)),
                  pl.BlockSpec((1, 64), lambda i: (0, 0)),
                  pl.BlockSpec((1, 64), lambda i: (0, 0))],
        out_specs=[pl.BlockSpec((BR, width), lambda i: (i, 0))] * nsplit,
        out_shape=[jax.ShapeDtypeStruct((n_pad, width), jnp.float32)] * nsplit)


def _tc_bnrelumax(n_nodes, nb):
    def kern(p, st, g, b, emb_out):
        i = pl.program_id(0)
        mu = st[0:1, :] * (1.0 / n_nodes)
        var = st[1:2, :] * (1.0 / n_nodes) - mu * mu
        scale = g[...] * lax.rsqrt(var + 1e-5)
        h = jnp.maximum((p[...] - mu) * scale + b[...], 0.0)
        h = jnp.where(_rowmask(i, n_nodes), h, -jnp.inf)
        bm = jnp.max(h, 0, keepdims=True)

        @pl.when(i == 0)
        def _():
            emb_out[...] = bm

        @pl.when(i > 0)
        def _():
            emb_out[...] = jnp.maximum(emb_out[...], bm)

    return pl.pallas_call(
        kern, grid=(nb,),
        in_specs=[pl.BlockSpec((BR, 64), lambda i: (i, 0)),
                  pl.BlockSpec((2, 64), lambda i: (0, 0)),
                  pl.BlockSpec((1, 64), lambda i: (0, 0)),
                  pl.BlockSpec((1, 64), lambda i: (0, 0))],
        out_specs=pl.BlockSpec((1, 64), lambda i: (0, 0)),
        out_shape=jax.ShapeDtypeStruct((1, 64), jnp.float32))


def _tc_joint():
    def kern(de, re_, w, b, out):
        j = jnp.concatenate([de[...], re_[...]], axis=1)
        out[...] = jnp.maximum(
            jnp.dot(j, w[...], preferred_element_type=jnp.float32) + b[...], 0.0)

    return pl.pallas_call(kern, out_shape=jax.ShapeDtypeStruct((1, 128),
                                                               jnp.float32))


def _pack_edges(ei, e_real, nchunks_pad, n_nodes):
    npad = nchunks_pad * CH - e_real
    pad = n_nodes + (jnp.arange(npad, dtype=jnp.int32) % 128)
    src = jnp.concatenate([ei[0], pad]).reshape(nchunks_pad, CH)
    dst = jnp.concatenate([ei[1], pad]).reshape(nchunks_pad, CH)
    return jnp.stack([src, dst], axis=1)


def kernel(dag_x, dag_edge_index, res_x, res_edge_index, dag_f1_Wl, dag_f1_Wr,
           dag_f1_b, dag_b1_Wl, dag_b1_Wr, dag_b1_b, dag_f2_Wl, dag_f2_Wr,
           dag_f2_b, dag_b2_Wl, dag_b2_Wr, dag_b2_b, dag_bn1_g, dag_bn1_b,
           dag_bn2_g, dag_bn2_b, res_c1_Wl, res_c1_Wr, res_c1_b, res_c2_Wl,
           res_c2_Wr, res_c2_b, res_bn1_g, res_bn1_b, res_bn2_g, res_bn2_b,
           joint_W, joint_b):
    f32 = jnp.float32
    xd = jnp.zeros((NDP, 16), f32).at[:N_D, :5].set(dag_x).at[:N_D, 5].set(1.0)
    xr = jnp.zeros((NRP, 16), f32).at[:N_R, :2].set(res_x).at[:N_R, 2].set(1.0)
    e_d = _pack_edges(dag_edge_index, E_D, DCHP, N_D)
    e_r = _pack_edges(res_edge_index, E_R, RCHP, N_R)

    z16 = jnp.zeros((16, 64), f32)
    wl1f = z16.at[:5].set(dag_f1_Wl)
    wl1b = z16.at[:5].set(dag_b1_Wl)
    wr1 = z16.at[:5].set(dag_f1_Wr + dag_b1_Wr)
    b1 = (dag_f1_b + dag_b1_b).reshape(1, 64)
    wr2 = dag_f2_Wr + dag_b2_Wr
    b2 = (dag_f2_b + dag_b2_b).reshape(1, 64)
    rwl1 = z16.at[:2].set(res_c1_Wl)
    rwr1 = z16.at[:2].set(res_c1_Wr)

    dphase = lambda fwd: dict(n_pad=NDP, fwd=fwd, ia=0, ib=0, ei=0, bpt=20,
                              seg16=True)
    rphase1 = dict(n_pad=NRP, fwd=True, ia=1, ib=1, ei=1, bpt=8, seg16=True)
    s1 = _sc_seg(16, 10, [dphase(True), dphase(False), rphase1], 2, 2)(
        xd, xr, e_d, e_r)

    p1, st1 = _tc_pre1(N_D, NB_D, 5, True, 0)(s1, s1, s1, s1, xd, wl1f, wl1b,
                                              wr1, b1)
    h0, h1, h2, h3 = _tc_bnrelu(N_D, NB_D, 4)(p1, st1, dag_bn1_g.reshape(1, 64),
                                              dag_bn1_b.reshape(1, 64))
    q1, rt1 = _tc_pre1(N_R, NB_R, 2, False, 4 * NB_D)(s1, s1, xr, rwl1, rwr1,
                                                      res_c1_b.reshape(1, 64))
    g0, g1, g2, g3 = _tc_bnrelu(N_R, NB_R, 4)(q1, rt1, res_bn1_g.reshape(1, 64),
                                              res_bn1_b.reshape(1, 64))

    d2 = lambda fwd, ia, ib: dict(n_pad=NDP, fwd=fwd, ia=ia, ib=ib, ei=0,
                                  bpt=50, seg16=False)
    r2 = lambda ia, ib: dict(n_pad=NRP, fwd=True, ia=ia, ib=ib, ei=1, bpt=20,
                             seg16=False)
    s2 = _sc_seg(16, 8, [d2(True, 0, 1), d2(True, 2, 3), d2(False, 0, 1),
                         d2(False, 2, 3), r2(4, 5), r2(6, 7)], 8, 2)(
        h0, h1, h2, h3, g0, g1, g2, g3, e_d, e_r)

    p2, st2 = _tc_pre2(N_D, NB_D, 5, True, 4, 0, 0)(
        s2, s2, s2, s2, s2, s2, s2, s2,
        h0, h1, h2, h3, s1, s1, s1, s1,
        dag_f2_Wl, dag_b2_Wl, wr2, b2)
    demb = _tc_bnrelumax(N_D, NB_D)(p2, st2, dag_bn2_g.reshape(1, 64),
                                    dag_bn2_b.reshape(1, 64))

    q2, rt2 = _tc_pre2(N_R, NB_R, 2, False, 4, 8 * NB_D, 4 * NB_D)(
        s2, s2, s2, s2, g0, g1, g2, g3, s1, s1, res_c2_Wl, res_c2_Wr,
        res_c2_b.reshape(1, 64))
    remb = _tc_bnrelumax(N_R, NB_R)(q2, rt2, res_bn2_g.reshape(1, 64),
                                    res_bn2_b.reshape(1, 64))

    out = _tc_joint()(demb, remb, joint_W, joint_b.reshape(1, 128))
    return out.reshape(128)


# speedup vs baseline: 1.2420x; 1.2420x over previous
import jax
import jax.numpy as jnp
from jax import lax
from jax.experimental import pallas as pl
from jax.experimental.pallas import tpu as pltpu
from jax.experimental.pallas import tpu_sc as plsc

N_D, E_D = 50000, 800000
N_R, E_R = 10000, 320000
H = 64
BR = 1024
NB_D, NB_R = 49, 10
NDP = NB_D * BR
NRP = NB_R * BR
CH = 128
NTILES = 16
DCHP = 6400
RCHP = 2560

_mesh = lambda: plsc.VectorSubcoreMesh(core_axis_name="c", subcore_axis_name="s",
                                       num_cores=2, num_subcores=16)
_SC_PARAMS = pltpu.CompilerParams(use_tc_tiling_on_sc=False)


def _zero_fill(slab_v, srows):
    z = jnp.zeros((16,), jnp.float32)
    width = slab_v.shape[1]

    def zb(i, _):
        for k in range(8):
            for c0 in range(0, width, 16):
                slab_v[i * 8 + k, c0:c0 + 16] = z
        return _

    lax.fori_loop(0, srows // 8, zb, None)


def _sc_seg(width, kb, phases, ntab, ne):
    max_pad = max(p["n_pad"] for p in phases)
    max_srows = max_pad // NTILES // 2
    out_rows = sum(2 * p["n_pad"] for p in phases)
    scratch = ([
        pltpu.VMEM((2, kb, 2, CH), jnp.int32),
        pltpu.VMEM((2, kb, CH, width), jnp.float32),
        pltpu.VMEM((max_srows, width), jnp.float32),
        pltpu.VMEM_SHARED((max_pad, width), jnp.float32),
    ] + [pltpu.SemaphoreType.DMA] * (kb + 3))

    def body(*args):
        tabs = args[:ntab]
        es = args[ntab:ntab + ne]
        out = args[ntab + ne]
        idx_v, rows_v, slab_v, acc = args[ntab + ne + 1:ntab + ne + 5]
        sems = args[ntab + ne + 5:]
        gsem, ssem, isem = sems[:kb], sems[kb], sems[kb + 1:]
        c = lax.axis_index("c")
        s = lax.axis_index("s")

        off = 0
        for ph in phases:
            n_pad, bpt = ph["n_pad"], ph["bpt"]
            gi, si = (0, 1) if ph["fwd"] else (1, 0)
            ta, tb = tabs[ph["ia"]], tabs[ph["ib"]]
            e_hbm = es[ph["ei"]]
            rows_per_tile = n_pad // NTILES
            srows = rows_per_tile // 2
            row0 = s * rows_per_tile
            _zero_fill(slab_v, srows)
            for h in range(2):
                pltpu.sync_copy(slab_v.at[pl.ds(0, srows)],
                                acc.at[pl.ds(row0 + h * srows, srows), :])
            plsc.subcore_barrier()
            if ph["seg16"]:
                base = (s * 2 + c) * bpt * kb
            else:
                base = s * bpt * kb

            def firegather(b, table):
                return [pltpu.async_copy(table.at[idx_v.at[b, k, gi]],
                                         rows_v.at[b, k], gsem[k])
                        for k in range(kb)]

            def firescatter(b):
                for k in range(kb):
                    pltpu.make_async_copy(ta.at[pl.ds(0, CH)],
                                          rows_v.at[b, k], gsem[k]).wait()
                    pltpu.async_copy(rows_v.at[b, k], acc.at[idx_v.at[b, k, si]],
                                     ssem, add=True)

            pltpu.async_copy(e_hbm.at[pl.ds(base, kb)], idx_v.at[0], isem[0])

            def pair(j, carry):
                for b in range(2):
                    t = j * 2 + b
                    pltpu.make_async_copy(e_hbm.at[pl.ds(base, kb)],
                                          idx_v.at[b], isem[b]).wait()
                    if ph["seg16"]:
                        firegather(b, ta)
                    else:
                        @pl.when(c == 0)
                        def _g0():
                            firegather(b, ta)

                        @pl.when(c == 1)
                        def _g1():
                            firegather(b, tb)

                    @pl.when(t >= 1)
                    def _drain():
                        for k in range(kb):
                            pltpu.make_async_copy(ta.at[pl.ds(0, CH)],
                                                  rows_v.at[1 - b, k],
                                                  ssem).wait()

                    @pl.when(t + 1 < bpt)
                    def _prefetch():
                        pltpu.async_copy(
                            e_hbm.at[pl.ds(base + (t + 1) * kb, kb)],
                            idx_v.at[1 - b], isem[1 - b])

                    firescatter(b)
                return carry

            lax.fori_loop(0, bpt // 2, pair, None)
            b_last = 1
            for k in range(kb):
                pltpu.make_async_copy(ta.at[pl.ds(0, CH)],
                                      rows_v.at[b_last, k], ssem).wait()
            plsc.subcore_barrier()
            obase = off + c * n_pad + row0
            for h in range(2):
                pltpu.sync_copy(acc.at[pl.ds(row0 + h * srows, srows), :],
                                slab_v.at[pl.ds(0, srows)])
                pltpu.sync_copy(slab_v.at[pl.ds(0, srows)],
                                out.at[pl.ds(obase + h * srows, srows), :])
            off += 2 * n_pad

    return pl.kernel(body,
                     out_type=jax.ShapeDtypeStruct((out_rows, width),
                                                   jnp.float32),
                     mesh=_mesh(), scratch_types=scratch,
                     compiler_params=_SC_PARAMS)


def _rowmask(i, n_nodes):
    rows = i * BR + lax.broadcasted_iota(jnp.int32, (BR, 1), 0)
    return rows < n_nodes


def _stats_accum(i, st_out, p):
    st = jnp.concatenate([jnp.sum(p, 0, keepdims=True),
                          jnp.sum(p * p, 0, keepdims=True)], 0)

    @pl.when(i == 0)
    def _():
        st_out[...] = st

    @pl.when(i > 0)
    def _():
        st_out[...] = st_out[...] + st


def _tc_pre1(n_nodes, nb, deg_col, two_dir, soff):
    def kern(*args):
        if two_dir:
            (sf0, sf1, sb0, sb1, x, wlf, wlb, wr, b, p_out, st_out) = args
        else:
            (sf0, sf1, x, wlf, wr, b, p_out, st_out) = args
        i = pl.program_id(0)
        sfb = sf0[...] + sf1[...]
        mf = sfb / jnp.maximum(sfb[:, deg_col:deg_col + 1], 1.0)
        p = jnp.dot(mf, wlf[...], preferred_element_type=jnp.float32)
        if two_dir:
            sbb = sb0[...] + sb1[...]
            mb = sbb / jnp.maximum(sbb[:, deg_col:deg_col + 1], 1.0)
            p = p + jnp.dot(mb, wlb[...], preferred_element_type=jnp.float32)
        p = p + jnp.dot(x[...], wr[...], preferred_element_type=jnp.float32)
        p = p + b[...]
        p = jnp.where(_rowmask(i, n_nodes), p, 0.0)
        p_out[...] = p
        _stats_accum(i, st_out, p)

    n_pad = nb * BR
    half = lambda j: pl.BlockSpec((BR, 16),
                                  lambda i, j=j: (i + soff + j * nb, 0))
    full16 = pl.BlockSpec((16, 64), lambda i: (0, 0))
    in_specs = [half(0), half(1)]
    if two_dir:
        in_specs += [half(2), half(3)]
    in_specs += [pl.BlockSpec((BR, 16), lambda i: (i, 0)), full16]
    if two_dir:
        in_specs += [full16]
    in_specs += [full16, pl.BlockSpec((1, 64), lambda i: (0, 0))]
    return pl.pallas_call(
        kern, grid=(nb,), in_specs=in_specs,
        out_specs=[pl.BlockSpec((BR, 64), lambda i: (i, 0)),
                   pl.BlockSpec((2, 64), lambda i: (0, 0))],
        out_shape=[jax.ShapeDtypeStruct((n_pad, 64), jnp.float32),
                   jax.ShapeDtypeStruct((2, 64), jnp.float32)])


def _tc_pre2(n_nodes, nb, deg_col, two_dir, nsplit, s2off, s1off):
    width = 64 // nsplit

    def kern(*args):
        args = list(args)
        s2f = [args.pop(0) for _ in range(nsplit)]
        s2b = [args.pop(0) for _ in range(nsplit)] if two_dir else None
        hs = [args.pop(0) for _ in range(nsplit)]
        sf0, sf1 = args.pop(0), args.pop(0)
        sb = (args.pop(0), args.pop(0)) if two_dir else None
        if two_dir:
            wlf, wlb, wr, b, p_out, st_out = args
        else:
            wlf, wr, b, p_out, st_out = args
        i = pl.program_id(0)
        degf = jnp.maximum(sf0[:, deg_col:deg_col + 1]
                           + sf1[:, deg_col:deg_col + 1], 1.0)
        m2f = jnp.concatenate([r[...] for r in s2f], axis=1) / degf
        p = jnp.dot(m2f, wlf[...], preferred_element_type=jnp.float32)
        if two_dir:
            degb = jnp.maximum(sb[0][:, deg_col:deg_col + 1]
                               + sb[1][:, deg_col:deg_col + 1], 1.0)
            m2b = jnp.concatenate([r[...] for r in s2b], axis=1) / degb
            p = p + jnp.dot(m2b, wlb[...], preferred_element_type=jnp.float32)
        hcat = jnp.concatenate([r[...] for r in hs], axis=1)
        p = p + jnp.dot(hcat, wr[...], preferred_element_type=jnp.float32)
        p = p + b[...]
        p = jnp.where(_rowmask(i, n_nodes), p, 0.0)
        p_out[...] = p
        _stats_accum(i, st_out, p)

    n_pad = nb * BR
    strip = lambda j: pl.BlockSpec((BR, width),
                                   lambda i, j=j: (i + s2off + j * nb, 0))
    s16 = lambda j: pl.BlockSpec((BR, 16),
                                 lambda i, j=j: (i + s1off + j * nb, 0))
    hstrip = pl.BlockSpec((BR, width), lambda i: (i, 0))
    full64 = pl.BlockSpec((64, 64), lambda i: (0, 0))
    in_specs = [strip(j) for j in range(nsplit)]
    if two_dir:
        in_specs += [strip(nsplit + j) for j in range(nsplit)]
    in_specs += [hstrip] * nsplit
    in_specs += [s16(0), s16(1)]
    if two_dir:
        in_specs += [s16(2), s16(3)]
    in_specs += [full64]
    if two_dir:
        in_specs += [full64]
    in_specs += [full64, pl.BlockSpec((1, 64), lambda i: (0, 0))]
    return pl.pallas_call(
        kern, grid=(nb,), in_specs=in_specs,
        out_specs=[pl.BlockSpec((BR, 64), lambda i: (i, 0)),
                   pl.BlockSpec((2, 64), lambda i: (0, 0))],
        out_shape=[jax.ShapeDtypeStruct((n_pad, 64), jnp.float32),
                   jax.ShapeDtypeStruct((2, 64), jnp.float32)])


def _tc_bnrelu(n_nodes, nb, nsplit):
    width = 64 // nsplit

    def kern(*args):
        p, st, g, b = args[:4]
        outs = args[4:]
        mu = st[0:1, :] * (1.0 / n_nodes)
        var = st[1:2, :] * (1.0 / n_nodes) - mu * mu
        scale = g[...] * lax.rsqrt(var + 1e-5)
        h = jnp.maximum((p[...] - mu) * scale + b[...], 0.0)
        for j, o in enumerate(outs):
            o[...] = h[:, j * width:(j + 1) * width]

    n_pad = nb * BR
    return pl.pallas_call(
        kern, grid=(nb,),
        in_specs=[pl.BlockSpec((BR, 64), lambda i: (i, 0)),
                  pl.BlockSpec((2, 64), lambda i: (0, 0)),
                  pl.BlockSpec((1, 64), lambda i: (0, 0)),
                  pl.BlockSpec((1, 64), lambda i: (0, 0))],
        out_specs=[pl.BlockSpec((BR, width), lambda i: (i, 0))] * nsplit,
        out_shape=[jax.ShapeDtypeStruct((n_pad, width), jnp.float32)] * nsplit)


def _tc_bnrelumax(n_nodes, nb):
    def kern(p, st, g, b, emb_out):
        i = pl.program_id(0)
        mu = st[0:1, :] * (1.0 / n_nodes)
        var = st[1:2, :] * (1.0 / n_nodes) - mu * mu
        scale = g[...] * lax.rsqrt(var + 1e-5)
        h = jnp.maximum((p[...] - mu) * scale + b[...], 0.0)
        h = jnp.where(_rowmask(i, n_nodes), h, -jnp.inf)
        bm = jnp.max(h, 0, keepdims=True)

        @pl.when(i == 0)
        def _():
            emb_out[...] = bm

        @pl.when(i > 0)
        def _():
            emb_out[...] = jnp.maximum(emb_out[...], bm)

    return pl.pallas_call(
        kern, grid=(nb,),
        in_specs=[pl.BlockSpec((BR, 64), lambda i: (i, 0)),
                  pl.BlockSpec((2, 64), lambda i: (0, 0)),
                  pl.BlockSpec((1, 64), lambda i: (0, 0)),
                  pl.BlockSpec((1, 64), lambda i: (0, 0))],
        out_specs=pl.BlockSpec((1, 64), lambda i: (0, 0)),
        out_shape=jax.ShapeDtypeStruct((1, 64), jnp.float32))


def _tc_joint():
    def kern(de, re_, w, b, out):
        j = jnp.concatenate([de[...], re_[...]], axis=1)
        out[...] = jnp.maximum(
            jnp.dot(j, w[...], preferred_element_type=jnp.float32) + b[...], 0.0)

    return pl.pallas_call(kern, out_shape=jax.ShapeDtypeStruct((1, 128),
                                                               jnp.float32))


def _pack_edges(ei, e_real, nchunks_pad, n_nodes):
    npad = nchunks_pad * CH - e_real
    pad = n_nodes + (jnp.arange(npad, dtype=jnp.int32) % 128)
    src = jnp.concatenate([ei[0], pad]).reshape(nchunks_pad, CH)
    dst = jnp.concatenate([ei[1], pad]).reshape(nchunks_pad, CH)
    return jnp.stack([src, dst], axis=1)


def kernel(dag_x, dag_edge_index, res_x, res_edge_index, dag_f1_Wl, dag_f1_Wr,
           dag_f1_b, dag_b1_Wl, dag_b1_Wr, dag_b1_b, dag_f2_Wl, dag_f2_Wr,
           dag_f2_b, dag_b2_Wl, dag_b2_Wr, dag_b2_b, dag_bn1_g, dag_bn1_b,
           dag_bn2_g, dag_bn2_b, res_c1_Wl, res_c1_Wr, res_c1_b, res_c2_Wl,
           res_c2_Wr, res_c2_b, res_bn1_g, res_bn1_b, res_bn2_g, res_bn2_b,
           joint_W, joint_b):
    f32 = jnp.float32
    xd = jnp.zeros((NDP, 16), f32).at[:N_D, :5].set(dag_x).at[:N_D, 5].set(1.0)
    xr = jnp.zeros((NRP, 16), f32).at[:N_R, :2].set(res_x).at[:N_R, 2].set(1.0)
    e_d = _pack_edges(dag_edge_index, E_D, DCHP, N_D)
    e_r = _pack_edges(res_edge_index, E_R, RCHP, N_R)

    z16 = jnp.zeros((16, 64), f32)
    wl1f = z16.at[:5].set(dag_f1_Wl)
    wl1b = z16.at[:5].set(dag_b1_Wl)
    wr1 = z16.at[:5].set(dag_f1_Wr + dag_b1_Wr)
    b1 = (dag_f1_b + dag_b1_b).reshape(1, 64)
    wr2 = dag_f2_Wr + dag_b2_Wr
    b2 = (dag_f2_b + dag_b2_b).reshape(1, 64)
    rwl1 = z16.at[:2].set(res_c1_Wl)
    rwr1 = z16.at[:2].set(res_c1_Wr)

    dphase = lambda fwd: dict(n_pad=NDP, fwd=fwd, ia=0, ib=0, ei=0, bpt=20,
                              seg16=True)
    s1 = _sc_seg(16, 10, [dphase(True), dphase(False)], 1, 1)(xd, e_d)
    p1, st1 = _tc_pre1(N_D, NB_D, 5, True, 0)(s1, s1, s1, s1, xd, wl1f, wl1b,
                                              wr1, b1)
    h0, h1, h2, h3 = _tc_bnrelu(N_D, NB_D, 4)(p1, st1, dag_bn1_g.reshape(1, 64),
                                              dag_bn1_b.reshape(1, 64))
    d2 = lambda fwd, ia, ib: dict(n_pad=NDP, fwd=fwd, ia=ia, ib=ib, ei=0,
                                  bpt=50, seg16=False)
    s2 = _sc_seg(16, 8, [d2(True, 0, 1), d2(True, 2, 3), d2(False, 0, 1),
                         d2(False, 2, 3)], 4, 1)(h0, h1, h2, h3, e_d)
    p2, st2 = _tc_pre2(N_D, NB_D, 5, True, 4, 0, 0)(
        s2, s2, s2, s2, s2, s2, s2, s2,
        h0, h1, h2, h3, s1, s1, s1, s1,
        dag_f2_Wl, dag_b2_Wl, wr2, b2)
    demb = _tc_bnrelumax(N_D, NB_D)(p2, st2, dag_bn2_g.reshape(1, 64),
                                    dag_bn2_b.reshape(1, 64))

    rs = _sc_seg(16, 8, [dict(n_pad=NRP, fwd=True, ia=0, ib=0, ei=0, bpt=10,
                              seg16=True)], 1, 1)(xr, e_r)
    q1, rt1 = _tc_pre1(N_R, NB_R, 2, False, 0)(rs, rs, xr, rwl1, rwr1,
                                               res_c1_b.reshape(1, 64))
    gl, gr = _tc_bnrelu(N_R, NB_R, 2)(q1, rt1, res_bn1_g.reshape(1, 64),
                                      res_bn1_b.reshape(1, 64))
    rs2 = _sc_seg(32, 8, [dict(n_pad=NRP, fwd=True, ia=0, ib=1, ei=0, bpt=20,
                               seg16=False)], 2, 1)(gl, gr, e_r)
    q2, rt2 = _tc_pre2(N_R, NB_R, 2, False, 2, 0, 0)(
        rs2, rs2, gl, gr, rs, rs, res_c2_Wl, res_c2_Wr,
        res_c2_b.reshape(1, 64))
    remb = _tc_bnrelumax(N_R, NB_R)(q2, rt2, res_bn2_g.reshape(1, 64),
                                    res_bn2_b.reshape(1, 64))

    out = _tc_joint()(demb, remb, joint_W, joint_b.reshape(1, 128))
    return out.reshape(128)


# speedup vs baseline: 1.2551x; 1.0105x over previous
import jax
import jax.numpy as jnp
from jax import lax
from jax.experimental import pallas as pl
from jax.experimental.pallas import tpu as pltpu
from jax.experimental.pallas import tpu_sc as plsc

N_D, E_D = 50000, 800000
N_R, E_R = 10000, 320000
H = 64
BR = 1024
NB_D, NB_R = 49, 10
NDP = NB_D * BR
NRP = NB_R * BR
CH = 128
NTILES = 16
DCHP = 6400
RCHP = 2560

_mesh = lambda: plsc.VectorSubcoreMesh(core_axis_name="c", subcore_axis_name="s",
                                       num_cores=2, num_subcores=16)
_SC_PARAMS = pltpu.CompilerParams(use_tc_tiling_on_sc=False)


def _zero_fill(slab_v, srows):
    z = jnp.zeros((16,), jnp.float32)
    width = slab_v.shape[1]

    def zb(i, _):
        for k in range(8):
            for c0 in range(0, width, 16):
                slab_v[i * 8 + k, c0:c0 + 16] = z
        return _

    lax.fori_loop(0, srows // 8, zb, None)


def _sc_seg(width, kb, phases, ntab, ne):
    max_pad = max(p["n_pad"] for p in phases)
    max_srows = max_pad // NTILES // 2
    out_rows = sum(2 * p["n_pad"] for p in phases)
    scratch = ([
        pltpu.VMEM((2, kb, 2, CH), jnp.int32),
        pltpu.VMEM((2, kb, CH, width), jnp.float32),
        pltpu.VMEM((max_srows, width), jnp.float32),
        pltpu.VMEM_SHARED((max_pad, width), jnp.float32),
    ] + [pltpu.SemaphoreType.DMA] * (kb + 3))

    def body(*args):
        tabs = args[:ntab]
        es = args[ntab:ntab + ne]
        out = args[ntab + ne]
        idx_v, rows_v, slab_v, acc = args[ntab + ne + 1:ntab + ne + 5]
        sems = args[ntab + ne + 5:]
        gsem, ssem, isem = sems[:kb], sems[kb], sems[kb + 1:]
        c = lax.axis_index("c")
        s = lax.axis_index("s")

        off = 0
        for ph in phases:
            n_pad, bpt = ph["n_pad"], ph["bpt"]
            gi, si = (0, 1) if ph["fwd"] else (1, 0)
            ta, tb = tabs[ph["ia"]], tabs[ph["ib"]]
            e_hbm = es[ph["ei"]]
            rows_per_tile = n_pad // NTILES
            srows = rows_per_tile // 2
            row0 = s * rows_per_tile
            _zero_fill(slab_v, srows)
            for h in range(2):
                pltpu.sync_copy(slab_v.at[pl.ds(0, srows)],
                                acc.at[pl.ds(row0 + h * srows, srows), :])
            plsc.subcore_barrier()
            if ph["seg16"]:
                base = (s * 2 + c) * bpt * kb
            else:
                base = s * bpt * kb

            def firegather(b, table):
                return [pltpu.async_copy(table.at[idx_v.at[b, k, gi]],
                                         rows_v.at[b, k], gsem[k])
                        for k in range(kb)]

            def firescatter(b):
                for k in range(kb):
                    pltpu.make_async_copy(ta.at[pl.ds(0, CH)],
                                          rows_v.at[b, k], gsem[k]).wait()
                    pltpu.async_copy(rows_v.at[b, k], acc.at[idx_v.at[b, k, si]],
                                     ssem, add=True)

            pltpu.async_copy(e_hbm.at[pl.ds(base, kb)], idx_v.at[0], isem[0])

            def pair(j, carry):
                for b in range(2):
                    t = j * 2 + b
                    pltpu.make_async_copy(e_hbm.at[pl.ds(base, kb)],
                                          idx_v.at[b], isem[b]).wait()
                    if ph["seg16"]:
                        firegather(b, ta)
                    else:
                        @pl.when(c == 0)
                        def _g0():
                            firegather(b, ta)

                        @pl.when(c == 1)
                        def _g1():
                            firegather(b, tb)

                    @pl.when(t >= 1)
                    def _drain():
                        for k in range(kb):
                            pltpu.make_async_copy(ta.at[pl.ds(0, CH)],
                                                  rows_v.at[1 - b, k],
                                                  ssem).wait()

                    @pl.when(t + 1 < bpt)
                    def _prefetch():
                        pltpu.async_copy(
                            e_hbm.at[pl.ds(base + (t + 1) * kb, kb)],
                            idx_v.at[1 - b], isem[1 - b])

                    firescatter(b)
                return carry

            lax.fori_loop(0, bpt // 2, pair, None)
            b_last = 1
            for k in range(kb):
                pltpu.make_async_copy(ta.at[pl.ds(0, CH)],
                                      rows_v.at[b_last, k], ssem).wait()
            plsc.subcore_barrier()
            obase = off + c * n_pad + row0
            for h in range(2):
                pltpu.sync_copy(acc.at[pl.ds(row0 + h * srows, srows), :],
                                slab_v.at[pl.ds(0, srows)])
                pltpu.sync_copy(slab_v.at[pl.ds(0, srows)],
                                out.at[pl.ds(obase + h * srows, srows), :])
            off += 2 * n_pad

    return pl.kernel(body,
                     out_type=jax.ShapeDtypeStruct((out_rows, width),
                                                   jnp.float32),
                     mesh=_mesh(), scratch_types=scratch,
                     compiler_params=_SC_PARAMS)


def _rowmask(i, n_nodes):
    rows = i * BR + lax.broadcasted_iota(jnp.int32, (BR, 1), 0)
    return rows < n_nodes


def _stats_accum(i, st_out, p):
    st = jnp.concatenate([jnp.sum(p, 0, keepdims=True),
                          jnp.sum(p * p, 0, keepdims=True)], 0)

    @pl.when(i == 0)
    def _():
        st_out[...] = st

    @pl.when(i > 0)
    def _():
        st_out[...] = st_out[...] + st


def _tc_pre1(n_nodes, nb, deg_col, two_dir, soff):
    def kern(*args):
        if two_dir:
            (sf0, sf1, sb0, sb1, x, wlf, wlb, wr, b, p_out, st_out) = args
        else:
            (sf0, sf1, x, wlf, wr, b, p_out, st_out) = args
        i = pl.program_id(0)
        sfb = sf0[...] + sf1[...]
        mf = sfb / jnp.maximum(sfb[:, deg_col:deg_col + 1], 1.0)
        p = jnp.dot(mf, wlf[...], preferred_element_type=jnp.float32)
        if two_dir:
            sbb = sb0[...] + sb1[...]
            mb = sbb / jnp.maximum(sbb[:, deg_col:deg_col + 1], 1.0)
            p = p + jnp.dot(mb, wlb[...], preferred_element_type=jnp.float32)
        p = p + jnp.dot(x[...], wr[...], preferred_element_type=jnp.float32)
        p = p + b[...]
        p = jnp.where(_rowmask(i, n_nodes), p, 0.0)
        p_out[...] = p
        _stats_accum(i, st_out, p)

    n_pad = nb * BR
    half = lambda j: pl.BlockSpec((BR, 16),
                                  lambda i, j=j: (i + soff + j * nb, 0))
    full16 = pl.BlockSpec((16, 64), lambda i: (0, 0))
    in_specs = [half(0), half(1)]
    if two_dir:
        in_specs += [half(2), half(3)]
    in_specs += [pl.BlockSpec((BR, 16), lambda i: (i, 0)), full16]
    if two_dir:
        in_specs += [full16]
    in_specs += [full16, pl.BlockSpec((1, 64), lambda i: (0, 0))]
    return pl.pallas_call(
        kern, grid=(nb,), in_specs=in_specs,
        out_specs=[pl.BlockSpec((BR, 64), lambda i: (i, 0)),
                   pl.BlockSpec((2, 64), lambda i: (0, 0))],
        out_shape=[jax.ShapeDtypeStruct((n_pad, 64), jnp.float32),
                   jax.ShapeDtypeStruct((2, 64), jnp.float32)])


def _tc_pre2(n_nodes, nb, deg_col, two_dir, nsplit, s2off, s1off):
    width = 64 // nsplit

    def kern(*args):
        args = list(args)
        s2f = [args.pop(0) for _ in range(nsplit)]
        s2b = [args.pop(0) for _ in range(nsplit)] if two_dir else None
        hs = [args.pop(0) for _ in range(nsplit)]
        sf0, sf1 = args.pop(0), args.pop(0)
        sb = (args.pop(0), args.pop(0)) if two_dir else None
        if two_dir:
            wlf, wlb, wr, b, p_out, st_out = args
        else:
            wlf, wr, b, p_out, st_out = args
        i = pl.program_id(0)
        degf = jnp.maximum(sf0[:, deg_col:deg_col + 1]
                           + sf1[:, deg_col:deg_col + 1], 1.0)
        m2f = jnp.concatenate([r[...] for r in s2f], axis=1) / degf
        p = jnp.dot(m2f, wlf[...], preferred_element_type=jnp.float32)
        if two_dir:
            degb = jnp.maximum(sb[0][:, deg_col:deg_col + 1]
                               + sb[1][:, deg_col:deg_col + 1], 1.0)
            m2b = jnp.concatenate([r[...] for r in s2b], axis=1) / degb
            p = p + jnp.dot(m2b, wlb[...], preferred_element_type=jnp.float32)
        hcat = jnp.concatenate([r[...] for r in hs], axis=1)
        p = p + jnp.dot(hcat, wr[...], preferred_element_type=jnp.float32)
        p = p + b[...]
        p = jnp.where(_rowmask(i, n_nodes), p, 0.0)
        p_out[...] = p
        _stats_accum(i, st_out, p)

    n_pad = nb * BR
    strip = lambda j: pl.BlockSpec((BR, width),
                                   lambda i, j=j: (i + s2off + j * nb, 0))
    s16 = lambda j: pl.BlockSpec((BR, 16),
                                 lambda i, j=j: (i + s1off + j * nb, 0))
    hstrip = pl.BlockSpec((BR, width), lambda i: (i, 0))
    full64 = pl.BlockSpec((64, 64), lambda i: (0, 0))
    in_specs = [strip(j) for j in range(nsplit)]
    if two_dir:
        in_specs += [strip(nsplit + j) for j in range(nsplit)]
    in_specs += [hstrip] * nsplit
    in_specs += [s16(0), s16(1)]
    if two_dir:
        in_specs += [s16(2), s16(3)]
    in_specs += [full64]
    if two_dir:
        in_specs += [full64]
    in_specs += [full64, pl.BlockSpec((1, 64), lambda i: (0, 0))]
    return pl.pallas_call(
        kern, grid=(nb,), in_specs=in_specs,
        out_specs=[pl.BlockSpec((BR, 64), lambda i: (i, 0)),
                   pl.BlockSpec((2, 64), lambda i: (0, 0))],
        out_shape=[jax.ShapeDtypeStruct((n_pad, 64), jnp.float32),
                   jax.ShapeDtypeStruct((2, 64), jnp.float32)])


def _tc_bnrelu(n_nodes, nb, nsplit):
    width = 64 // nsplit

    def kern(*args):
        p, st, g, b = args[:4]
        outs = args[4:]
        mu = st[0:1, :] * (1.0 / n_nodes)
        var = st[1:2, :] * (1.0 / n_nodes) - mu * mu
        scale = g[...] * lax.rsqrt(var + 1e-5)
        h = jnp.maximum((p[...] - mu) * scale + b[...], 0.0)
        for j, o in enumerate(outs):
            o[...] = h[:, j * width:(j + 1) * width]

    n_pad = nb * BR
    return pl.pallas_call(
        kern, grid=(nb,),
        in_specs=[pl.BlockSpec((BR, 64), lambda i: (i, 0)),
                  pl.BlockSpec((2, 64), lambda i: (0, 0)),
                  pl.BlockSpec((1, 64), lambda i: (0, 0)),
                  pl.BlockSpec((1, 64), lambda i: (0, 0))],
        out_specs=[pl.BlockSpec((BR, width), lambda i: (i, 0))] * nsplit,
        out_shape=[jax.ShapeDtypeStruct((n_pad, width), jnp.float32)] * nsplit)


def _tc_bnrelumax(n_nodes, nb):
    def kern(p, st, g, b, emb_out):
        i = pl.program_id(0)
        mu = st[0:1, :] * (1.0 / n_nodes)
        var = st[1:2, :] * (1.0 / n_nodes) - mu * mu
        scale = g[...] * lax.rsqrt(var + 1e-5)
        h = jnp.maximum((p[...] - mu) * scale + b[...], 0.0)
        h = jnp.where(_rowmask(i, n_nodes), h, -jnp.inf)
        bm = jnp.max(h, 0, keepdims=True)

        @pl.when(i == 0)
        def _():
            emb_out[...] = bm

        @pl.when(i > 0)
        def _():
            emb_out[...] = jnp.maximum(emb_out[...], bm)

    return pl.pallas_call(
        kern, grid=(nb,),
        in_specs=[pl.BlockSpec((BR, 64), lambda i: (i, 0)),
                  pl.BlockSpec((2, 64), lambda i: (0, 0)),
                  pl.BlockSpec((1, 64), lambda i: (0, 0)),
                  pl.BlockSpec((1, 64), lambda i: (0, 0))],
        out_specs=pl.BlockSpec((1, 64), lambda i: (0, 0)),
        out_shape=jax.ShapeDtypeStruct((1, 64), jnp.float32))


def _tc_joint():
    def kern(de, re_, w, b, out):
        j = jnp.concatenate([de[...], re_[...]], axis=1)
        out[...] = jnp.maximum(
            jnp.dot(j, w[...], preferred_element_type=jnp.float32) + b[...], 0.0)

    return pl.pallas_call(kern, out_shape=jax.ShapeDtypeStruct((1, 128),
                                                               jnp.float32))


def _pack_edges(ei, e_real, nchunks_pad, n_nodes):
    npad = nchunks_pad * CH - e_real
    pad = n_nodes + (jnp.arange(npad, dtype=jnp.int32) % 128)
    src = jnp.concatenate([ei[0], pad]).reshape(nchunks_pad, CH)
    dst = jnp.concatenate([ei[1], pad]).reshape(nchunks_pad, CH)
    return jnp.stack([src, dst], axis=1)


def kernel(dag_x, dag_edge_index, res_x, res_edge_index, dag_f1_Wl, dag_f1_Wr,
           dag_f1_b, dag_b1_Wl, dag_b1_Wr, dag_b1_b, dag_f2_Wl, dag_f2_Wr,
           dag_f2_b, dag_b2_Wl, dag_b2_Wr, dag_b2_b, dag_bn1_g, dag_bn1_b,
           dag_bn2_g, dag_bn2_b, res_c1_Wl, res_c1_Wr, res_c1_b, res_c2_Wl,
           res_c2_Wr, res_c2_b, res_bn1_g, res_bn1_b, res_bn2_g, res_bn2_b,
           joint_W, joint_b):
    f32 = jnp.float32
    xd = jnp.zeros((NDP, 16), f32).at[:N_D, :5].set(dag_x).at[:N_D, 5].set(1.0)
    xr = jnp.zeros((NRP, 16), f32).at[:N_R, :2].set(res_x).at[:N_R, 2].set(1.0)
    e_d = _pack_edges(dag_edge_index, E_D, DCHP, N_D)
    e_r = _pack_edges(res_edge_index, E_R, RCHP, N_R)

    z16 = jnp.zeros((16, 64), f32)
    wl1f = z16.at[:5].set(dag_f1_Wl)
    wl1b = z16.at[:5].set(dag_b1_Wl)
    wr1 = z16.at[:5].set(dag_f1_Wr + dag_b1_Wr)
    b1 = (dag_f1_b + dag_b1_b).reshape(1, 64)
    wr2 = dag_f2_Wr + dag_b2_Wr
    b2 = (dag_f2_b + dag_b2_b).reshape(1, 64)
    rwl1 = z16.at[:2].set(res_c1_Wl)
    rwr1 = z16.at[:2].set(res_c1_Wr)

    dphase = lambda fwd: dict(n_pad=NDP, fwd=fwd, ia=0, ib=0, ei=0, bpt=20,
                              seg16=True)
    s1 = _sc_seg(16, 10, [dphase(True), dphase(False)], 1, 1)(xd, e_d)
    p1, st1 = _tc_pre1(N_D, NB_D, 5, True, 0)(s1, s1, s1, s1, xd, wl1f, wl1b,
                                              wr1, b1)
    h0, h1, h2, h3 = _tc_bnrelu(N_D, NB_D, 4)(p1, st1, dag_bn1_g.reshape(1, 64),
                                              dag_bn1_b.reshape(1, 64))
    d2 = lambda fwd, ia, ib: dict(n_pad=NDP, fwd=fwd, ia=ia, ib=ib, ei=0,
                                  bpt=40, seg16=False)
    s2 = _sc_seg(16, 10, [d2(True, 0, 1), d2(True, 2, 3), d2(False, 0, 1),
                          d2(False, 2, 3)], 4, 1)(h0, h1, h2, h3, e_d)
    p2, st2 = _tc_pre2(N_D, NB_D, 5, True, 4, 0, 0)(
        s2, s2, s2, s2, s2, s2, s2, s2,
        h0, h1, h2, h3, s1, s1, s1, s1,
        dag_f2_Wl, dag_b2_Wl, wr2, b2)
    demb = _tc_bnrelumax(N_D, NB_D)(p2, st2, dag_bn2_g.reshape(1, 64),
                                    dag_bn2_b.reshape(1, 64))

    rs = _sc_seg(16, 10, [dict(n_pad=NRP, fwd=True, ia=0, ib=0, ei=0, bpt=8,
                               seg16=True)], 1, 1)(xr, e_r)
    q1, rt1 = _tc_pre1(N_R, NB_R, 2, False, 0)(rs, rs, xr, rwl1, rwr1,
                                               res_c1_b.reshape(1, 64))
    gl, gr = _tc_bnrelu(N_R, NB_R, 2)(q1, rt1, res_bn1_g.reshape(1, 64),
                                      res_bn1_b.reshape(1, 64))
    rs2 = _sc_seg(32, 10, [dict(n_pad=NRP, fwd=True, ia=0, ib=1, ei=0, bpt=16,
                                seg16=False)], 2, 1)(gl, gr, e_r)
    q2, rt2 = _tc_pre2(N_R, NB_R, 2, False, 2, 0, 0)(
        rs2, rs2, gl, gr, rs, rs, res_c2_Wl, res_c2_Wr,
        res_c2_b.reshape(1, 64))
    remb = _tc_bnrelumax(N_R, NB_R)(q2, rt2, res_bn2_g.reshape(1, 64),
                                    res_bn2_b.reshape(1, 64))

    out = _tc_joint()(demb, remb, joint_W, joint_b.reshape(1, 128))
    return out.reshape(128)
